# Initial kernel scaffold; baseline (speedup 1.0000x reference)
import functools

import jax
import jax.numpy as jnp
from jax import lax
from jax.experimental import pallas as pl
from jax.experimental.pallas import tpu as pltpu
from jax.experimental.pallas import tpu_sc as plsc

B, N, D, H, MAX_K = 4, 2048, 768, 256, 20
KPAD = 32
BLKN = 512
NB = N // BLKN


def _score_sum_kernel(tok_ref, wp1_ref, bp1_ref, wp2_ref, scores_ref, sum_ref):
    j = pl.program_id(1)
    t = tok_ref[0]
    h = jnp.maximum(
        jnp.dot(t, wp1_ref[...], preferred_element_type=jnp.float32)
        + bp1_ref[...], 0.0)
    s = jnp.dot(h, wp2_ref[...], preferred_element_type=jnp.float32)
    scores_ref[0, 0] = s
    partial = jnp.sum(t, axis=0, keepdims=True)

    @pl.when(j == 0)
    def _():
        sum_ref[0] = partial

    @pl.when(j != 0)
    def _():
        sum_ref[0] += partial


def _scores_and_sums(tokens, W_p1, b_p1, W_p2):
    scores4, tsum = pl.pallas_call(
        _score_sum_kernel,
        grid=(B, NB),
        in_specs=[
            pl.BlockSpec((1, BLKN, D), lambda b, j: (b, j, 0)),
            pl.BlockSpec((D, H), lambda b, j: (0, 0)),
            pl.BlockSpec((1, H), lambda b, j: (0, 0)),
            pl.BlockSpec((H, 1), lambda b, j: (0, 0)),
        ],
        out_specs=[
            pl.BlockSpec((1, 1, BLKN, 1), lambda b, j: (b, j, 0, 0)),
            pl.BlockSpec((1, 1, D), lambda b, j: (b, 0, 0)),
        ],
        out_shape=[
            jax.ShapeDtypeStruct((B, NB, BLKN, 1), jnp.float32),
            jax.ShapeDtypeStruct((B, 1, D), jnp.float32),
        ],
    )(tokens, W_p1, b_p1.reshape(1, H), W_p2)
    return scores4.reshape(B, N), tsum.reshape(B, D)


def _sc_topk_gather_body(scores_hbm, tokens_hbm, out_hbm, sc_v, idx_v, rows_v, sem):
    wid = lax.axis_index("s") * 2 + lax.axis_index("c")

    @pl.when(wid < B)
    def _():
        b = wid
        pltpu.sync_copy(scores_hbm.at[b], sc_v)
        base = b * N
        lane = lax.iota(jnp.int32, 16)
        zeros16 = jnp.zeros((16,), jnp.int32)
        neg = jnp.float32(-3.0e38)
        negv = jnp.zeros((16,), jnp.float32) + neg
        idx0 = zeros16 + base
        idx1 = zeros16 + base

        for r in range(MAX_K):
            def body(j, carry):
                vmax, cchunk = carry
                v = sc_v[pl.ds(j * 16, 16)]
                upd = v > vmax
                vmax = jnp.where(upd, v, vmax)
                cchunk = jnp.where(upd, zeros16 + j, cchunk)
                return vmax, cchunk

            vmax, cchunk = lax.fori_loop(
                0, N // 16, body, (negv, zeros16))
            m = jnp.max(vmax)
            cand = cchunk * 16 + lane + jnp.where(
                vmax == m, 0, jnp.int32(1 << 24))
            sel = jnp.min(cand)
            gi = base + sel
            if r < 16:
                idx0 = jnp.where(lane == (r % 16), gi, idx0)
            else:
                idx1 = jnp.where(lane == (r % 16), gi, idx1)
            off = sel & jnp.int32(-16)
            l0 = sel & jnp.int32(15)
            v = sc_v[pl.ds(off, 16)]
            sc_v[pl.ds(off, 16)] = jnp.where(lane == l0, neg, v)

        idx_v[pl.ds(0, 16)] = idx0
        idx_v[pl.ds(16, 16)] = idx1
        pltpu.async_copy(tokens_hbm.at[idx_v], rows_v, sem).wait()
        pltpu.sync_copy(rows_v, out_hbm.at[b])


def _topk_gather(scores, tokens_flat):
    mesh = plsc.VectorSubcoreMesh(core_axis_name="c", subcore_axis_name="s")
    fn = functools.partial(
        pl.kernel,
        out_type=jax.ShapeDtypeStruct((B, KPAD, D), jnp.float32),
        mesh=mesh,
        scratch_types=[
            pltpu.VMEM((N,), jnp.float32),
            pltpu.VMEM((KPAD,), jnp.int32),
            pltpu.VMEM((KPAD, D), jnp.float32),
            pltpu.SemaphoreType.DMA,
        ],
    )(_sc_topk_gather_body)
    return fn(scores, tokens_flat)


def _final_kernel(sum_ref, g_ref, we, be, wa1, ba1, wa2, ba2, wk1, bk1,
                  wk2, bk2, wr1, br1, wr2, br2, wf1, bf1, wf2, bf2, out_ref):
    ts = sum_ref[...]
    ri = ts * (1.0 / N)
    feat = jnp.maximum(
        jnp.dot(ri, we[...], preferred_element_type=jnp.float32) + be[...], 0.0)
    ah = jnp.maximum(
        jnp.dot(feat, wa1[...], preferred_element_type=jnp.float32) + ba1[...], 0.0)
    alogit = jnp.dot(ah, wa2[...], preferred_element_type=jnp.float32) + ba2[...]
    alpha = 1.0 / (1.0 + jnp.exp(-alogit))
    kh = jnp.maximum(
        jnp.dot(feat, wk1[...], preferred_element_type=jnp.float32) + bk1[...], 0.0)
    kx = jnp.dot(kh, wk2[...], preferred_element_type=jnp.float32) + bk2[...]
    kraw = jnp.maximum(kx, 0.0) + jnp.log1p(jnp.exp(-jnp.abs(kx)))
    kkf = jnp.clip(jnp.round(kraw), 1.0, float(MAX_K))
    ta = jnp.maximum(1.0, jnp.floor(alpha * kkf))

    g = g_ref[...]
    g2 = g.reshape(B * KPAD, D)
    rh = jnp.maximum(
        jnp.dot(g2, wr1[...], preferred_element_type=jnp.float32) + br1[...], 0.0)
    rr = jnp.dot(rh, wr2[...], preferred_element_type=jnp.float32) + br2[...]
    rr = rr.reshape(B, KPAD, D)

    pos = lax.broadcasted_iota(jnp.float32, (B, KPAD), 1)
    mask = (pos < ta).astype(jnp.float32)[:, :, None]
    refined_sum = jnp.sum(rr * mask, axis=1)
    sel_sum = jnp.sum(g * mask, axis=1)
    pooled = (ts - sel_sum) / (float(N) - ta)
    fm = (refined_sum + pooled) / (ta + 1.0)
    fh = jnp.maximum(
        jnp.dot(fm, wf1[...], preferred_element_type=jnp.float32) + bf1[...], 0.0)
    out_ref[...] = jnp.dot(fh, wf2[...], preferred_element_type=jnp.float32) + bf2[...]


def _make_spec(shape):
    nd = len(shape)
    return pl.BlockSpec(shape, lambda *_, __nd=nd: (0,) * __nd)


def _final(token_sum, gathered, W_enc, b_enc, W_a1, b_a1, W_a2, b_a2,
           W_k1, b_k1, W_k2, b_k2, W_r1, b_r1, W_r2, b_r2,
           W_f1, b_f1, W_f2, b_f2):
    args = [token_sum, gathered,
            W_enc, b_enc.reshape(1, -1), W_a1, b_a1.reshape(1, -1),
            W_a2, b_a2.reshape(1, -1), W_k1, b_k1.reshape(1, -1),
            W_k2, b_k2.reshape(1, -1), W_r1, b_r1.reshape(1, -1),
            W_r2, b_r2.reshape(1, -1), W_f1, b_f1.reshape(1, -1),
            W_f2, b_f2.reshape(1, -1)]
    return pl.pallas_call(
        _final_kernel,
        in_specs=[_make_spec(a.shape) for a in args],
        out_specs=pl.BlockSpec((B, D), lambda: (0, 0)),
        out_shape=jax.ShapeDtypeStruct((B, D), jnp.float32),
    )(*args)


def kernel(tokens, W_enc, b_enc, W_a1, b_a1, W_a2, b_a2, W_k1, b_k1,
           W_k2, b_k2, W_p1, b_p1, W_p2, b_p2, W_r1, b_r1, W_r2, b_r2,
           W_f1, b_f1, W_f2, b_f2):
    scores, token_sum = _scores_and_sums(tokens, W_p1, b_p1, W_p2)
    gathered = _topk_gather(scores, tokens.reshape(B * N, D))
    return _final(token_sum, gathered, W_enc, b_enc, W_a1, b_a1, W_a2, b_a2,
                  W_k1, b_k1, W_k2, b_k2, W_r1, b_r1, W_r2, b_r2,
                  W_f1, b_f1, W_f2, b_f2)


# speedup vs baseline: 1.4596x; 1.4596x over previous
import functools

import jax
import jax.numpy as jnp
from jax import lax
from jax.experimental import pallas as pl
from jax.experimental.pallas import tpu as pltpu
from jax.experimental.pallas import tpu_sc as plsc

B, N, D, H, MAX_K = 4, 2048, 768, 256, 20
KPAD = 32
BLKN = 512
NB = N // BLKN


def _score_sum_kernel(tok_ref, wp1_ref, bp1_ref, wp2_ref, scores_ref, sum_ref):
    j = pl.program_id(1)
    t = tok_ref[0]
    h = jnp.maximum(
        jnp.dot(t, wp1_ref[...], preferred_element_type=jnp.float32)
        + bp1_ref[...], 0.0)
    s = jnp.dot(h, wp2_ref[...], preferred_element_type=jnp.float32)
    scores_ref[0, 0] = s
    partial = jnp.sum(t, axis=0, keepdims=True)

    @pl.when(j == 0)
    def _():
        sum_ref[0] = partial

    @pl.when(j != 0)
    def _():
        sum_ref[0] += partial


def _scores_and_sums(tokens, W_p1, b_p1, W_p2):
    scores4, tsum = pl.pallas_call(
        _score_sum_kernel,
        grid=(B, NB),
        in_specs=[
            pl.BlockSpec((1, BLKN, D), lambda b, j: (b, j, 0)),
            pl.BlockSpec((D, H), lambda b, j: (0, 0)),
            pl.BlockSpec((1, H), lambda b, j: (0, 0)),
            pl.BlockSpec((H, 1), lambda b, j: (0, 0)),
        ],
        out_specs=[
            pl.BlockSpec((1, 1, BLKN, 1), lambda b, j: (b, j, 0, 0)),
            pl.BlockSpec((1, 1, D), lambda b, j: (b, 0, 0)),
        ],
        out_shape=[
            jax.ShapeDtypeStruct((B, NB, BLKN, 1), jnp.float32),
            jax.ShapeDtypeStruct((B, 1, D), jnp.float32),
        ],
    )(tokens, W_p1, b_p1.reshape(1, H), W_p2)
    return scores4.reshape(B, N), tsum.reshape(B, D)


def _sc_topk_gather_body(scores_hbm, tokens_hbm, out_hbm, sc_v,
                         idx_v, rows_v, sem):
    wid = lax.axis_index("s") * 2 + lax.axis_index("c")

    @pl.when(wid < B)
    def _():
        b = wid
        pltpu.sync_copy(scores_hbm.at[b], sc_v)
        base = b * N
        lane = lax.iota(jnp.int32, 16)
        zeros16 = jnp.zeros((16,), jnp.int32)
        neg = jnp.float32(-3.0e38)
        negv = jnp.zeros((16,), jnp.float32) + neg
        idx0 = zeros16 + base
        idx1 = zeros16 + base

        for r in range(MAX_K):
            def body(j, carry):
                vmax, cchunk = carry
                v = sc_v[pl.ds(j * 16, 16)]
                upd = v > vmax
                vmax = jnp.where(upd, v, vmax)
                cchunk = jnp.where(upd, zeros16 + j, cchunk)
                return vmax, cchunk

            vmax, cchunk = lax.fori_loop(
                0, N // 16, body, (negv, zeros16))
            vred = vmax
            cred = cchunk * 16 + lane
            for step in (8, 4, 2, 1):
                perm = lane ^ step
                vp = vred.at[perm].get(mode="promise_in_bounds")
                cp = cred.at[perm].get(mode="promise_in_bounds")
                swap = (vp > vred) | ((vp == vred) & (cp < cred))
                vred = jnp.where(swap, vp, vred)
                cred = jnp.where(swap, cp, cred)
            sel = cred[0]
            gi = base + sel
            if r < 16:
                idx0 = jnp.where(lane == (r % 16), gi, idx0)
            else:
                idx1 = jnp.where(lane == (r % 16), gi, idx1)
            off = sel & jnp.int32(-16)
            l0 = sel & jnp.int32(15)
            v = sc_v[pl.ds(off, 16)]
            sc_v[pl.ds(off, 16)] = jnp.where(lane == l0, neg, v)

        idx_v[pl.ds(0, 16)] = idx0
        idx_v[pl.ds(16, 16)] = idx1
        pltpu.async_copy(tokens_hbm.at[idx_v], rows_v, sem).wait()
        pltpu.sync_copy(rows_v, out_hbm.at[b])


def _topk_gather(scores, tokens_flat):
    mesh = plsc.VectorSubcoreMesh(core_axis_name="c", subcore_axis_name="s")
    fn = functools.partial(
        pl.kernel,
        out_type=jax.ShapeDtypeStruct((B, KPAD, D), jnp.float32),
        mesh=mesh,
        scratch_types=[
            pltpu.VMEM((N,), jnp.float32),
            pltpu.VMEM((KPAD,), jnp.int32),
            pltpu.VMEM((KPAD, D), jnp.float32),
            pltpu.SemaphoreType.DMA,
        ],
    )(_sc_topk_gather_body)
    return fn(scores, tokens_flat)


def _final_kernel(sum_ref, g_ref, we, be, wa1, ba1, wa2, ba2, wk1, bk1,
                  wk2, bk2, wr1, br1, wr2, br2, wf1, bf1, wf2, bf2, out_ref):
    ts = sum_ref[...]
    ri = ts * (1.0 / N)
    feat = jnp.maximum(
        jnp.dot(ri, we[...], preferred_element_type=jnp.float32) + be[...], 0.0)
    ah = jnp.maximum(
        jnp.dot(feat, wa1[...], preferred_element_type=jnp.float32) + ba1[...], 0.0)
    alogit = jnp.dot(ah, wa2[...], preferred_element_type=jnp.float32) + ba2[...]
    alpha = 1.0 / (1.0 + jnp.exp(-alogit))
    kh = jnp.maximum(
        jnp.dot(feat, wk1[...], preferred_element_type=jnp.float32) + bk1[...], 0.0)
    kx = jnp.dot(kh, wk2[...], preferred_element_type=jnp.float32) + bk2[...]
    kraw = jnp.maximum(kx, 0.0) + jnp.log1p(jnp.exp(-jnp.abs(kx)))
    kkf = jnp.clip(jnp.round(kraw), 1.0, float(MAX_K))
    ta = jnp.maximum(1.0, jnp.floor(alpha * kkf))

    g = g_ref[...]
    g2 = g.reshape(B * KPAD, D)
    rh = jnp.maximum(
        jnp.dot(g2, wr1[...], preferred_element_type=jnp.float32) + br1[...], 0.0)
    rr = jnp.dot(rh, wr2[...], preferred_element_type=jnp.float32) + br2[...]
    rr = rr.reshape(B, KPAD, D)

    pos = lax.broadcasted_iota(jnp.int32, (B, KPAD), 1).astype(jnp.float32)
    mask = (pos < ta).astype(jnp.float32)[:, :, None]
    refined_sum = jnp.sum(rr * mask, axis=1)
    sel_sum = jnp.sum(g * mask, axis=1)
    pooled = (ts - sel_sum) / (float(N) - ta)
    fm = (refined_sum + pooled) / (ta + 1.0)
    fh = jnp.maximum(
        jnp.dot(fm, wf1[...], preferred_element_type=jnp.float32) + bf1[...], 0.0)
    out_ref[...] = jnp.dot(fh, wf2[...], preferred_element_type=jnp.float32) + bf2[...]


def _make_spec(shape):
    nd = len(shape)
    return pl.BlockSpec(shape, lambda *_, __nd=nd: (0,) * __nd)


def _final(token_sum, gathered, W_enc, b_enc, W_a1, b_a1, W_a2, b_a2,
           W_k1, b_k1, W_k2, b_k2, W_r1, b_r1, W_r2, b_r2,
           W_f1, b_f1, W_f2, b_f2):
    args = [token_sum, gathered,
            W_enc, b_enc.reshape(1, -1), W_a1, b_a1.reshape(1, -1),
            W_a2, b_a2.reshape(1, -1), W_k1, b_k1.reshape(1, -1),
            W_k2, b_k2.reshape(1, -1), W_r1, b_r1.reshape(1, -1),
            W_r2, b_r2.reshape(1, -1), W_f1, b_f1.reshape(1, -1),
            W_f2, b_f2.reshape(1, -1)]
    return pl.pallas_call(
        _final_kernel,
        in_specs=[_make_spec(a.shape) for a in args],
        out_specs=pl.BlockSpec((B, D), lambda: (0, 0)),
        out_shape=jax.ShapeDtypeStruct((B, D), jnp.float32),
    )(*args)


def kernel(tokens, W_enc, b_enc, W_a1, b_a1, W_a2, b_a2, W_k1, b_k1,
           W_k2, b_k2, W_p1, b_p1, W_p2, b_p2, W_r1, b_r1, W_r2, b_r2,
           W_f1, b_f1, W_f2, b_f2):
    scores, token_sum = _scores_and_sums(tokens, W_p1, b_p1, W_p2)
    gathered = _topk_gather(scores, tokens.reshape(B * N, D))
    return _final(token_sum, gathered, W_enc, b_enc, W_a1, b_a1, W_a2, b_a2,
                  W_k1, b_k1, W_k2, b_k2, W_r1, b_r1, W_r2, b_r2,
                  W_f1, b_f1, W_f2, b_f2)


# speedup vs baseline: 1.7554x; 1.2027x over previous
import functools

import jax
import jax.numpy as jnp
from jax import lax
from jax.experimental import pallas as pl
from jax.experimental.pallas import tpu as pltpu
from jax.experimental.pallas import tpu_sc as plsc

B, N, D, H, MAX_K = 4, 2048, 768, 256, 20
KPAD = 32
BLKN = 512
NB = N // BLKN


def _score_sum_kernel(tok_ref, wp1_ref, bp1_ref, wp2_ref, scores_ref, sum_ref):
    j = pl.program_id(1)
    t = tok_ref[0]
    h = jnp.maximum(
        jnp.dot(t, wp1_ref[...], preferred_element_type=jnp.float32)
        + bp1_ref[...], 0.0)
    s = jnp.dot(h, wp2_ref[...], preferred_element_type=jnp.float32)
    scores_ref[0, 0] = s
    partial = jnp.sum(t, axis=0, keepdims=True)

    @pl.when(j == 0)
    def _():
        sum_ref[0] = partial

    @pl.when(j != 0)
    def _():
        sum_ref[0] += partial


def _scores_and_sums(tokens, W_p1, b_p1, W_p2):
    scores4, tsum = pl.pallas_call(
        _score_sum_kernel,
        grid=(B, NB),
        in_specs=[
            pl.BlockSpec((1, BLKN, D), lambda b, j: (b, j, 0)),
            pl.BlockSpec((D, H), lambda b, j: (0, 0)),
            pl.BlockSpec((1, H), lambda b, j: (0, 0)),
            pl.BlockSpec((H, 1), lambda b, j: (0, 0)),
        ],
        out_specs=[
            pl.BlockSpec((1, 1, BLKN, 1), lambda b, j: (b, j, 0, 0)),
            pl.BlockSpec((1, 1, D), lambda b, j: (b, 0, 0)),
        ],
        out_shape=[
            jax.ShapeDtypeStruct((B, NB, BLKN, 1), jnp.float32),
            jax.ShapeDtypeStruct((B, 1, D), jnp.float32),
        ],
    )(tokens, W_p1, b_p1.reshape(1, H), W_p2)
    return scores4.reshape(B, N), tsum.reshape(B, D)


TPB = 8
CHUNK = N // TPB
SLOTS = KPAD
MERGE = TPB * SLOTS


def _sc_topk_gather_body(scores_hbm, tokens_hbm, out_hbm, sc_v, mv, mi,
                         stage_v, stage_i, idx_v, rows_v, shared_v, shared_i,
                         sem):
    c = lax.axis_index("c")
    s = lax.axis_index("s")
    bl = s // TPB
    b = c * 2 + bl
    t = s % TPB
    lane = lax.iota(jnp.int32, 16)
    zeros16 = jnp.zeros((16,), jnp.int32)
    neg = jnp.float32(-3.0e38)
    negv = jnp.zeros((16,), jnp.float32) + neg

    pltpu.sync_copy(scores_hbm.at[b, pl.ds(t * CHUNK, CHUNK)], sc_v)
    jbase = t * CHUNK

    def round1(r, carry):
        c0, c1, i0, i1 = carry
        vmax, vidx = negv, zeros16
        for j in range(CHUNK // 16):
            v = sc_v[pl.ds(j * 16, 16)]
            lin = zeros16 + (jbase + j * 16) + lane
            take = (v > vmax) | ((v == vmax) & (lin < vidx))
            vmax = jnp.where(take, v, vmax)
            vidx = jnp.where(take, lin, vidx)
        for step in (8, 4, 2, 1):
            vp = vmax.at[lane ^ step].get(mode="promise_in_bounds")
            ip = vidx.at[lane ^ step].get(mode="promise_in_bounds")
            swap = (vp > vmax) | ((vp == vmax) & (ip < vidx))
            vmax = jnp.where(swap, vp, vmax)
            vidx = jnp.where(swap, ip, vidx)
        hit0 = lane == jnp.where(r < 16, r, 99)
        hit1 = lane == jnp.where(r >= 16, r - 16, 99)
        c0 = jnp.where(hit0, vmax, c0)
        i0 = jnp.where(hit0, vidx, i0)
        c1 = jnp.where(hit1, vmax, c1)
        i1 = jnp.where(hit1, vidx, i1)
        loc = vidx[0] - jbase
        off = loc & jnp.int32(-16)
        l0 = loc & jnp.int32(15)
        vv = sc_v[pl.ds(off, 16)]
        sc_v[pl.ds(off, 16)] = jnp.where(lane == l0, neg, vv)
        return c0, c1, i0, i1

    c0, c1, i0, i1 = lax.fori_loop(
        0, MAX_K, round1, (negv, negv, zeros16, zeros16))

    stage_v[pl.ds(0, 16)] = c0
    stage_v[pl.ds(16, 16)] = c1
    stage_i[pl.ds(0, 16)] = i0
    stage_i[pl.ds(16, 16)] = i1
    pltpu.sync_copy(stage_v, shared_v.at[bl, pl.ds(t * SLOTS, SLOTS)])
    pltpu.sync_copy(stage_i, shared_i.at[bl, pl.ds(t * SLOTS, SLOTS)])
    plsc.subcore_barrier()

    @pl.when(t == 0)
    def _():
        pltpu.sync_copy(shared_v.at[bl], mv)
        pltpu.sync_copy(shared_i.at[bl], mi)
        base = b * N

        def round2(r, carry):
            idx0, idx1 = carry
            vmax, vidx, bpos = negv, zeros16, zeros16
            for j in range(MERGE // 16):
                v = mv[pl.ds(j * 16, 16)]
                i = mi[pl.ds(j * 16, 16)]
                take = (v > vmax) | ((v == vmax) & (i < vidx))
                vmax = jnp.where(take, v, vmax)
                vidx = jnp.where(take, i, vidx)
                bpos = jnp.where(take, zeros16 + j * 16 + lane, bpos)
            for step in (8, 4, 2, 1):
                vp = vmax.at[lane ^ step].get(mode="promise_in_bounds")
                ip = vidx.at[lane ^ step].get(mode="promise_in_bounds")
                pp = bpos.at[lane ^ step].get(mode="promise_in_bounds")
                swap = (vp > vmax) | ((vp == vmax) & (ip < vidx))
                vmax = jnp.where(swap, vp, vmax)
                vidx = jnp.where(swap, ip, vidx)
                bpos = jnp.where(swap, pp, bpos)
            gi = base + vidx
            hit0 = lane == jnp.where(r < 16, r, 99)
            hit1 = lane == jnp.where(r >= 16, r - 16, 99)
            idx0 = jnp.where(hit0, gi, idx0)
            idx1 = jnp.where(hit1, gi, idx1)
            p = bpos[0]
            off = p & jnp.int32(-16)
            l0 = p & jnp.int32(15)
            vv = mv[pl.ds(off, 16)]
            mv[pl.ds(off, 16)] = jnp.where(lane == l0, neg, vv)
            return idx0, idx1

        idx0, idx1 = lax.fori_loop(
            0, MAX_K, round2, (zeros16 + base, zeros16 + base))
        idx_v[pl.ds(0, 16)] = idx0
        idx_v[pl.ds(16, 16)] = idx1
        pltpu.async_copy(tokens_hbm.at[idx_v], rows_v, sem).wait()
        pltpu.sync_copy(rows_v, out_hbm.at[b])


def _topk_gather(scores, tokens_flat):
    mesh = plsc.VectorSubcoreMesh(core_axis_name="c", subcore_axis_name="s")
    fn = functools.partial(
        pl.kernel,
        out_type=jax.ShapeDtypeStruct((B, KPAD, D), jnp.float32),
        mesh=mesh,
        scratch_types=[
            pltpu.VMEM((CHUNK,), jnp.float32),
            pltpu.VMEM((MERGE,), jnp.float32),
            pltpu.VMEM((MERGE,), jnp.int32),
            pltpu.VMEM((SLOTS,), jnp.float32),
            pltpu.VMEM((SLOTS,), jnp.int32),
            pltpu.VMEM((KPAD,), jnp.int32),
            pltpu.VMEM((KPAD, D), jnp.float32),
            pltpu.VMEM_SHARED((2, MERGE), jnp.float32),
            pltpu.VMEM_SHARED((2, MERGE), jnp.int32),
            pltpu.SemaphoreType.DMA,
        ],
    )(_sc_topk_gather_body)
    return fn(scores, tokens_flat)


def _final_kernel(sum_ref, g_ref, we, be, wa1, ba1, wa2, ba2, wk1, bk1,
                  wk2, bk2, wr1, br1, wr2, br2, wf1, bf1, wf2, bf2, out_ref):
    ts = sum_ref[...]
    ri = ts * (1.0 / N)
    feat = jnp.maximum(
        jnp.dot(ri, we[...], preferred_element_type=jnp.float32) + be[...], 0.0)
    ah = jnp.maximum(
        jnp.dot(feat, wa1[...], preferred_element_type=jnp.float32) + ba1[...], 0.0)
    alogit = jnp.dot(ah, wa2[...], preferred_element_type=jnp.float32) + ba2[...]
    alpha = 1.0 / (1.0 + jnp.exp(-alogit))
    kh = jnp.maximum(
        jnp.dot(feat, wk1[...], preferred_element_type=jnp.float32) + bk1[...], 0.0)
    kx = jnp.dot(kh, wk2[...], preferred_element_type=jnp.float32) + bk2[...]
    kraw = jnp.maximum(kx, 0.0) + jnp.log1p(jnp.exp(-jnp.abs(kx)))
    kkf = jnp.clip(jnp.round(kraw), 1.0, float(MAX_K))
    ta = jnp.maximum(1.0, jnp.floor(alpha * kkf))

    g = g_ref[...]
    g2 = g.reshape(B * KPAD, D)
    rh = jnp.maximum(
        jnp.dot(g2, wr1[...], preferred_element_type=jnp.float32) + br1[...], 0.0)
    rr = jnp.dot(rh, wr2[...], preferred_element_type=jnp.float32) + br2[...]
    rr = rr.reshape(B, KPAD, D)

    pos = lax.broadcasted_iota(jnp.int32, (B, KPAD), 1).astype(jnp.float32)
    mask = (pos < ta).astype(jnp.float32)[:, :, None]
    refined_sum = jnp.sum(rr * mask, axis=1)
    sel_sum = jnp.sum(g * mask, axis=1)
    pooled = (ts - sel_sum) / (float(N) - ta)
    fm = (refined_sum + pooled) / (ta + 1.0)
    fh = jnp.maximum(
        jnp.dot(fm, wf1[...], preferred_element_type=jnp.float32) + bf1[...], 0.0)
    out_ref[...] = jnp.dot(fh, wf2[...], preferred_element_type=jnp.float32) + bf2[...]


def _make_spec(shape):
    nd = len(shape)
    return pl.BlockSpec(shape, lambda *_, __nd=nd: (0,) * __nd)


def _final(token_sum, gathered, W_enc, b_enc, W_a1, b_a1, W_a2, b_a2,
           W_k1, b_k1, W_k2, b_k2, W_r1, b_r1, W_r2, b_r2,
           W_f1, b_f1, W_f2, b_f2):
    args = [token_sum, gathered,
            W_enc, b_enc.reshape(1, -1), W_a1, b_a1.reshape(1, -1),
            W_a2, b_a2.reshape(1, -1), W_k1, b_k1.reshape(1, -1),
            W_k2, b_k2.reshape(1, -1), W_r1, b_r1.reshape(1, -1),
            W_r2, b_r2.reshape(1, -1), W_f1, b_f1.reshape(1, -1),
            W_f2, b_f2.reshape(1, -1)]
    return pl.pallas_call(
        _final_kernel,
        in_specs=[_make_spec(a.shape) for a in args],
        out_specs=pl.BlockSpec((B, D), lambda: (0, 0)),
        out_shape=jax.ShapeDtypeStruct((B, D), jnp.float32),
    )(*args)


def kernel(tokens, W_enc, b_enc, W_a1, b_a1, W_a2, b_a2, W_k1, b_k1,
           W_k2, b_k2, W_p1, b_p1, W_p2, b_p2, W_r1, b_r1, W_r2, b_r2,
           W_f1, b_f1, W_f2, b_f2):
    scores, token_sum = _scores_and_sums(tokens, W_p1, b_p1, W_p2)
    gathered = _topk_gather(scores, tokens.reshape(B * N, D))
    return _final(token_sum, gathered, W_enc, b_enc, W_a1, b_a1, W_a2, b_a2,
                  W_k1, b_k1, W_k2, b_k2, W_r1, b_r1, W_r2, b_r2,
                  W_f1, b_f1, W_f2, b_f2)


# speedup vs baseline: 2.0419x; 1.1632x over previous
import functools

import jax
import jax.numpy as jnp
from jax import lax
from jax.experimental import pallas as pl
from jax.experimental.pallas import tpu as pltpu
from jax.experimental.pallas import tpu_sc as plsc

B, N, D, H, MAX_K = 4, 2048, 768, 256, 20
KPAD = 32
BLKN = 2048
NB = N // BLKN


def _score_sum_kernel(tok_ref, wp1_ref, bp1_ref, wp2_ref, scores_ref, sum_ref):
    j = pl.program_id(1)
    t = tok_ref[0]
    h = jnp.maximum(
        jnp.dot(t, wp1_ref[...], preferred_element_type=jnp.float32)
        + bp1_ref[...], 0.0)
    s = jnp.dot(h, wp2_ref[...], preferred_element_type=jnp.float32)
    scores_ref[0, 0] = s
    partial = jnp.sum(t, axis=0, keepdims=True)

    @pl.when(j == 0)
    def _():
        sum_ref[0] = partial

    @pl.when(j != 0)
    def _():
        sum_ref[0] += partial


def _scores_and_sums(tokens, W_p1, b_p1, W_p2):
    scores4, tsum = pl.pallas_call(
        _score_sum_kernel,
        grid=(B, NB),
        in_specs=[
            pl.BlockSpec((1, BLKN, D), lambda b, j: (b, j, 0)),
            pl.BlockSpec((D, H), lambda b, j: (0, 0)),
            pl.BlockSpec((1, H), lambda b, j: (0, 0)),
            pl.BlockSpec((H, 1), lambda b, j: (0, 0)),
        ],
        out_specs=[
            pl.BlockSpec((1, 1, BLKN, 1), lambda b, j: (b, j, 0, 0)),
            pl.BlockSpec((1, 1, D), lambda b, j: (b, 0, 0)),
        ],
        out_shape=[
            jax.ShapeDtypeStruct((B, NB, BLKN, 1), jnp.float32),
            jax.ShapeDtypeStruct((B, 1, D), jnp.float32),
        ],
        compiler_params=pltpu.CompilerParams(
            dimension_semantics=("parallel", "arbitrary")),
    )(tokens, W_p1, b_p1.reshape(1, H), W_p2)
    return scores4.reshape(B, N), tsum.reshape(B, D)


TPB = 8
CHUNK = N // TPB
SLOTS = KPAD
MERGE = TPB * SLOTS


def _sc_topk_gather_body(scores_hbm, tokens_hbm, out_hbm, sc_v, mv, mi,
                         stage_v, stage_i, idx_v, rows_v, shared_v, shared_i,
                         sem):
    c = lax.axis_index("c")
    s = lax.axis_index("s")
    bl = s // TPB
    b = c * 2 + bl
    t = s % TPB
    lane = lax.iota(jnp.int32, 16)
    zeros16 = jnp.zeros((16,), jnp.int32)
    neg = jnp.float32(-3.0e38)
    negv = jnp.zeros((16,), jnp.float32) + neg

    pltpu.sync_copy(scores_hbm.at[b, pl.ds(t * CHUNK, CHUNK)], sc_v)
    jbase = t * CHUNK

    def round1(r, carry):
        c0, c1, i0, i1 = carry
        vmax, vidx = negv, zeros16
        for j in range(CHUNK // 16):
            v = sc_v[pl.ds(j * 16, 16)]
            lin = zeros16 + (jbase + j * 16) + lane
            take = (v > vmax) | ((v == vmax) & (lin < vidx))
            vmax = jnp.where(take, v, vmax)
            vidx = jnp.where(take, lin, vidx)
        for step in (8, 4, 2, 1):
            vp = vmax.at[lane ^ step].get(mode="promise_in_bounds")
            ip = vidx.at[lane ^ step].get(mode="promise_in_bounds")
            swap = (vp > vmax) | ((vp == vmax) & (ip < vidx))
            vmax = jnp.where(swap, vp, vmax)
            vidx = jnp.where(swap, ip, vidx)
        hit0 = lane == jnp.where(r < 16, r, 99)
        hit1 = lane == jnp.where(r >= 16, r - 16, 99)
        c0 = jnp.where(hit0, vmax, c0)
        i0 = jnp.where(hit0, vidx, i0)
        c1 = jnp.where(hit1, vmax, c1)
        i1 = jnp.where(hit1, vidx, i1)
        loc = vidx[0] - jbase
        off = loc & jnp.int32(-16)
        l0 = loc & jnp.int32(15)
        vv = sc_v[pl.ds(off, 16)]
        sc_v[pl.ds(off, 16)] = jnp.where(lane == l0, neg, vv)
        return c0, c1, i0, i1

    c0, c1, i0, i1 = lax.fori_loop(
        0, MAX_K, round1, (negv, negv, zeros16, zeros16))

    stage_v[pl.ds(0, 16)] = c0
    stage_v[pl.ds(16, 16)] = c1
    stage_i[pl.ds(0, 16)] = i0
    stage_i[pl.ds(16, 16)] = i1
    pltpu.sync_copy(stage_v, shared_v.at[bl, pl.ds(t * SLOTS, SLOTS)])
    pltpu.sync_copy(stage_i, shared_i.at[bl, pl.ds(t * SLOTS, SLOTS)])
    plsc.subcore_barrier()

    @pl.when(t == 0)
    def _():
        pltpu.sync_copy(shared_v.at[bl], mv)
        pltpu.sync_copy(shared_i.at[bl], mi)
        base = b * N

        def round2(r, carry):
            idx0, idx1 = carry
            vmax, vidx, bpos = negv, zeros16, zeros16
            for j in range(MERGE // 16):
                v = mv[pl.ds(j * 16, 16)]
                i = mi[pl.ds(j * 16, 16)]
                take = (v > vmax) | ((v == vmax) & (i < vidx))
                vmax = jnp.where(take, v, vmax)
                vidx = jnp.where(take, i, vidx)
                bpos = jnp.where(take, zeros16 + j * 16 + lane, bpos)
            for step in (8, 4, 2, 1):
                vp = vmax.at[lane ^ step].get(mode="promise_in_bounds")
                ip = vidx.at[lane ^ step].get(mode="promise_in_bounds")
                pp = bpos.at[lane ^ step].get(mode="promise_in_bounds")
                swap = (vp > vmax) | ((vp == vmax) & (ip < vidx))
                vmax = jnp.where(swap, vp, vmax)
                vidx = jnp.where(swap, ip, vidx)
                bpos = jnp.where(swap, pp, bpos)
            gi = base + vidx
            hit0 = lane == jnp.where(r < 16, r, 99)
            hit1 = lane == jnp.where(r >= 16, r - 16, 99)
            idx0 = jnp.where(hit0, gi, idx0)
            idx1 = jnp.where(hit1, gi, idx1)
            p = bpos[0]
            off = p & jnp.int32(-16)
            l0 = p & jnp.int32(15)
            vv = mv[pl.ds(off, 16)]
            mv[pl.ds(off, 16)] = jnp.where(lane == l0, neg, vv)
            return idx0, idx1

        idx0, idx1 = lax.fori_loop(
            0, MAX_K, round2, (zeros16 + base, zeros16 + base))
        idx_v[pl.ds(0, 16)] = idx0
        idx_v[pl.ds(16, 16)] = idx1
        pltpu.async_copy(tokens_hbm.at[idx_v], rows_v, sem).wait()
        pltpu.sync_copy(rows_v, out_hbm.at[b])


def _topk_gather(scores, tokens_flat):
    mesh = plsc.VectorSubcoreMesh(core_axis_name="c", subcore_axis_name="s")
    fn = functools.partial(
        pl.kernel,
        out_type=jax.ShapeDtypeStruct((B, KPAD, D), jnp.float32),
        mesh=mesh,
        scratch_types=[
            pltpu.VMEM((CHUNK,), jnp.float32),
            pltpu.VMEM((MERGE,), jnp.float32),
            pltpu.VMEM((MERGE,), jnp.int32),
            pltpu.VMEM((SLOTS,), jnp.float32),
            pltpu.VMEM((SLOTS,), jnp.int32),
            pltpu.VMEM((KPAD,), jnp.int32),
            pltpu.VMEM((KPAD, D), jnp.float32),
            pltpu.VMEM_SHARED((2, MERGE), jnp.float32),
            pltpu.VMEM_SHARED((2, MERGE), jnp.int32),
            pltpu.SemaphoreType.DMA,
        ],
    )(_sc_topk_gather_body)
    return fn(scores, tokens_flat)


def _final_kernel(sum_ref, g_ref, we, be, wa1, ba1, wa2, ba2, wk1, bk1,
                  wk2, bk2, wr1, br1, wr2, br2, wf1, bf1, wf2, bf2, out_ref):
    ts = sum_ref[...]
    ri = ts * (1.0 / N)
    feat = jnp.maximum(
        jnp.dot(ri, we[...], preferred_element_type=jnp.float32) + be[...], 0.0)
    ah = jnp.maximum(
        jnp.dot(feat, wa1[...], preferred_element_type=jnp.float32) + ba1[...], 0.0)
    alogit = jnp.dot(ah, wa2[...], preferred_element_type=jnp.float32) + ba2[...]
    alpha = 1.0 / (1.0 + jnp.exp(-alogit))
    kh = jnp.maximum(
        jnp.dot(feat, wk1[...], preferred_element_type=jnp.float32) + bk1[...], 0.0)
    kx = jnp.dot(kh, wk2[...], preferred_element_type=jnp.float32) + bk2[...]
    kraw = jnp.maximum(kx, 0.0) + jnp.log1p(jnp.exp(-jnp.abs(kx)))
    kkf = jnp.clip(jnp.round(kraw), 1.0, float(MAX_K))
    ta = jnp.maximum(1.0, jnp.floor(alpha * kkf))

    g = g_ref[...]
    g2 = g.reshape(B * KPAD, D)
    rh = jnp.maximum(
        jnp.dot(g2, wr1[...], preferred_element_type=jnp.float32) + br1[...], 0.0)
    rr = jnp.dot(rh, wr2[...], preferred_element_type=jnp.float32) + br2[...]
    rr = rr.reshape(B, KPAD, D)

    pos = lax.broadcasted_iota(jnp.int32, (B, KPAD), 1).astype(jnp.float32)
    mask = (pos < ta).astype(jnp.float32)[:, :, None]
    refined_sum = jnp.sum(rr * mask, axis=1)
    sel_sum = jnp.sum(g * mask, axis=1)
    pooled = (ts - sel_sum) / (float(N) - ta)
    fm = (refined_sum + pooled) / (ta + 1.0)
    fh = jnp.maximum(
        jnp.dot(fm, wf1[...], preferred_element_type=jnp.float32) + bf1[...], 0.0)
    out_ref[...] = jnp.dot(fh, wf2[...], preferred_element_type=jnp.float32) + bf2[...]


def _make_spec(shape):
    nd = len(shape)
    return pl.BlockSpec(shape, lambda *_, __nd=nd: (0,) * __nd)


def _final(token_sum, gathered, W_enc, b_enc, W_a1, b_a1, W_a2, b_a2,
           W_k1, b_k1, W_k2, b_k2, W_r1, b_r1, W_r2, b_r2,
           W_f1, b_f1, W_f2, b_f2):
    args = [token_sum, gathered,
            W_enc, b_enc.reshape(1, -1), W_a1, b_a1.reshape(1, -1),
            W_a2, b_a2.reshape(1, -1), W_k1, b_k1.reshape(1, -1),
            W_k2, b_k2.reshape(1, -1), W_r1, b_r1.reshape(1, -1),
            W_r2, b_r2.reshape(1, -1), W_f1, b_f1.reshape(1, -1),
            W_f2, b_f2.reshape(1, -1)]
    return pl.pallas_call(
        _final_kernel,
        in_specs=[_make_spec(a.shape) for a in args],
        out_specs=pl.BlockSpec((B, D), lambda: (0, 0)),
        out_shape=jax.ShapeDtypeStruct((B, D), jnp.float32),
    )(*args)


def kernel(tokens, W_enc, b_enc, W_a1, b_a1, W_a2, b_a2, W_k1, b_k1,
           W_k2, b_k2, W_p1, b_p1, W_p2, b_p2, W_r1, b_r1, W_r2, b_r2,
           W_f1, b_f1, W_f2, b_f2):
    scores, token_sum = _scores_and_sums(tokens, W_p1, b_p1, W_p2)
    gathered = _topk_gather(scores, tokens.reshape(B * N, D))
    return _final(token_sum, gathered, W_enc, b_enc, W_a1, b_a1, W_a2, b_a2,
                  W_k1, b_k1, W_k2, b_k2, W_r1, b_r1, W_r2, b_r2,
                  W_f1, b_f1, W_f2, b_f2)
